# baseline (speedup 1.0000x reference)
import functools

import jax
import jax.numpy as jnp
from jax import lax
from jax.experimental import pallas as pl
from jax.experimental.pallas import tpu as pltpu
from jax.experimental.pallas import tpu_sc as plsc

SEQ = 2048
D = 768
WDIM = 20
NWIN = 8
NSPAN = NWIN * SEQ - (NWIN * (NWIN - 1)) // 2
OUTD = 2 * D + WDIM
OFFS = [SEQ * w - (w * (w - 1)) // 2 for w in range(NWIN + 1)]
BUCKETS = [1, 2, 3, 4, 5, 5, 6, 7]
STRIP_ROWS = 16384
SR = 64
RT = 256
XPAD = SEQ + RT + 8


def _sc_stage():
    info = plsc.get_sparse_core_info()
    nc = info.num_cores

    mesh = plsc.VectorSubcoreMesh(core_axis_name="c", subcore_axis_name="s")

    @functools.partial(
        pl.kernel,
        mesh=mesh,
        out_type=(
            jax.ShapeDtypeStruct((NSPAN, OUTD), jnp.float32),
            jax.ShapeDtypeStruct((STRIP_ROWS, 32), jnp.float32),
        ),
        scratch_types=[
            pltpu.VMEM((SR, 32), jnp.float32),
            pltpu.VMEM((1, WDIM), jnp.float32),
            pltpu.SemaphoreType.DMA,
        ],
        compiler_params=pltpu.CompilerParams(use_tc_tiling_on_sc=False),
    )
    def k(x_hbm, swe_hbm, swe20_hbm, pre_hbm, strip_hbm, wsb, wrow, fill_sem):
        cid = lax.axis_index("c")
        sid = lax.axis_index("s")
        wid = sid * nc + cid
        wi = wid // 4
        q = wid % 4
        n = SEQ - wi
        off = SEQ * wi - (wi * (wi - 1)) // 2
        b = wi + 1 - (wi >= 5).astype(jnp.int32)

        fills = [
            pltpu.make_async_copy(
                swe_hbm.at[pl.ds(b, 1), :],
                wsb.at[pl.ds(r, 1), :], fill_sem)
            for r in range(SR)
        ]
        for cp in fills:
            cp.start()
        pltpu.sync_copy(swe20_hbm.at[pl.ds(b, 1), :], wrow)

        hcnt = (-off) % 8
        tcnt = (off + n) % 8

        @pl.when(q == 0)
        def _patch():
            for kk in range(7):
                @pl.when(kk < hcnt)
                def _head():
                    r = off + kk
                    j = kk
                    pltpu.sync_copy(x_hbm.at[pl.ds(j, 1), :],
                                    pre_hbm.at[pl.ds(r, 1), pl.ds(0, D)])
                    pltpu.sync_copy(x_hbm.at[pl.ds(j + wi, 1), :],
                                    pre_hbm.at[pl.ds(r, 1), pl.ds(D, D)])
                    pltpu.sync_copy(wrow,
                                    pre_hbm.at[pl.ds(r, 1), pl.ds(2 * D, WDIM)])

                @pl.when(kk < tcnt)
                def _tail():
                    r = off + n - 1 - kk
                    j = n - 1 - kk
                    pltpu.sync_copy(x_hbm.at[pl.ds(j, 1), :],
                                    pre_hbm.at[pl.ds(r, 1), pl.ds(0, D)])
                    pltpu.sync_copy(x_hbm.at[pl.ds(j + wi, 1), :],
                                    pre_hbm.at[pl.ds(r, 1), pl.ds(D, D)])
                    pltpu.sync_copy(wrow,
                                    pre_hbm.at[pl.ds(r, 1), pl.ds(2 * D, WDIM)])

        for cp in fills:
            cp.wait()
        for t in range(8):
            j0 = jnp.minimum((q * 8 + t) * SR, n - SR)
            pltpu.sync_copy(wsb, strip_hbm.at[pl.ds(off + j0, SR), :])

    return k


def _tc_stage():
    def body(xp_ref, strip_ref, pre_ref, out_ref, obuf_a, obuf_b, sem_a, sem_b):
        i = pl.program_id(0)
        widx = i // 8
        t = i % 8

        def make_branch(w):
            off = OFFS[w]
            n = SEQ - w
            d = (-off) % 8
            e = (d + w) % 8
            h = off + d
            tw = (off + n) & ~7

            def f(tt):
                r0 = jnp.minimum(h + tt * RT, tw - RT)
                j0 = r0 - off
                ja = pl.multiple_of(j0 - d, 8)
                a_blk = xp_ref[pl.ds(ja, RT + 8), :]
                a_rows = (pltpu.roll(a_blk, RT + 8 - d, 0) if d else a_blk)[0:RT]
                jea = pl.multiple_of(j0 + w - e, 8)
                b_blk = xp_ref[pl.ds(jea, RT + 8), :]
                b_rows = (pltpu.roll(b_blk, RT + 8 - e, 0) if e else b_blk)[0:RT]
                return a_rows, b_rows, r0

            return f

        a_rows, b_rows, r0 = lax.switch(
            widx, [make_branch(w) for w in range(NWIN)], t)
        r0 = pl.multiple_of(r0, 8)
        s_rows = strip_ref[pl.ds(r0, RT), :]

        def do_step(obuf, sem):
            @pl.when(i >= 2)
            def _drain_prev():
                pltpu.make_async_copy(
                    obuf, out_ref.at[pl.ds(r0, RT), :], sem).wait()

            obuf[:, pl.ds(0, D)] = a_rows
            obuf[:, pl.ds(D, D)] = b_rows
            obuf[:, pl.ds(2 * D, WDIM)] = s_rows[:, 0:WDIM]
            pltpu.make_async_copy(
                obuf, out_ref.at[pl.ds(r0, RT), :], sem).start()

        par = t % 2

        @pl.when(par == 0)
        def _even():
            do_step(obuf_a, sem_a)

        @pl.when(par == 1)
        def _odd():
            do_step(obuf_b, sem_b)

        @pl.when(i == NWIN * 8 - 1)
        def _final_drain():
            pltpu.make_async_copy(
                obuf_a, out_ref.at[pl.ds(r0, RT), :], sem_a).wait()
            pltpu.make_async_copy(
                obuf_b, out_ref.at[pl.ds(r0, RT), :], sem_b).wait()

    return pl.pallas_call(
        body,
        grid=(NWIN * 8,),
        in_specs=[
            pl.BlockSpec((XPAD, D), lambda i: (0, 0)),
            pl.BlockSpec((STRIP_ROWS, 32), lambda i: (0, 0)),
            pl.BlockSpec(memory_space=pl.ANY),
        ],
        out_specs=pl.BlockSpec(memory_space=pl.ANY),
        out_shape=jax.ShapeDtypeStruct((NSPAN, OUTD), jnp.float32),
        scratch_shapes=[
            pltpu.VMEM((RT, OUTD), jnp.float32),
            pltpu.VMEM((RT, OUTD), jnp.float32),
            pltpu.SemaphoreType.DMA,
            pltpu.SemaphoreType.DMA,
        ],
        input_output_aliases={2: 0},
    )


def kernel(x, span_width_embedding, batch_max_seq_len):
    del batch_max_seq_len
    x2 = x.reshape(SEQ, D)
    swe_pad = (
        jnp.zeros((span_width_embedding.shape[0], 32), span_width_embedding.dtype)
        .at[:, :WDIM]
        .set(span_width_embedding)
    )
    pre, strip = _sc_stage()(x2, swe_pad, span_width_embedding)
    xp = jnp.pad(x2, ((0, XPAD - SEQ), (0, 0)))
    out = _tc_stage()(xp, strip, pre)
    return out.reshape(1, NSPAN, OUTD)


# speedup vs baseline: 1.0302x; 1.0302x over previous
import functools

import jax
import jax.numpy as jnp
from jax import lax
from jax.experimental import pallas as pl
from jax.experimental.pallas import tpu as pltpu
from jax.experimental.pallas import tpu_sc as plsc

SEQ = 2048
D = 768
WDIM = 20
NWIN = 8
NSPAN = NWIN * SEQ - (NWIN * (NWIN - 1)) // 2
OUTD = 2 * D + WDIM
OFFS = [SEQ * w - (w * (w - 1)) // 2 for w in range(NWIN + 1)]
BUCKETS = [1, 2, 3, 4, 5, 5, 6, 7]
BR = 256
NBLK = (NSPAN + BR - 1) // BR
LD = BR + 8
XPAD = SEQ + LD + 16


def _sc_lookup():
    info = plsc.get_sparse_core_info()
    nc = info.num_cores

    mesh = plsc.VectorSubcoreMesh(core_axis_name="c", subcore_axis_name="s")

    @functools.partial(
        pl.kernel,
        mesh=mesh,
        out_type=jax.ShapeDtypeStruct((NWIN, 128), jnp.float32),
        compiler_params=pltpu.CompilerParams(use_tc_tiling_on_sc=False),
    )
    def k(swe_hbm, tab_hbm):
        cid = lax.axis_index("c")
        sid = lax.axis_index("s")
        wid = sid * nc + cid

        @pl.when(wid < NWIN)
        def _lookup():
            b = wid + 1 - (wid >= 5).astype(jnp.int32)
            pltpu.sync_copy(swe_hbm.at[pl.ds(b, 1), :],
                            tab_hbm.at[pl.ds(wid, 1), :])

    return k


def _win_of(a):
    w = jnp.int32(0)
    for kk in range(1, NWIN):
        w = w + (a >= OFFS[kk]).astype(jnp.int32)
    return w


def _tc_assemble():
    def body(xp_ref, tab_ref, out_ref):
        i = pl.program_id(0)
        a = i * BR

        def single(w):
            off = OFFS[w]
            d = (-off) % 8
            e = (d + w) % 8

            def f():
                j0 = a - off
                ja = pl.multiple_of(j0 - d, 8)
                a_blk = xp_ref[pl.ds(ja, LD), :]
                a_rows = (pltpu.roll(a_blk, LD - d, 0) if d else a_blk)[0:BR]
                jea = pl.multiple_of(j0 + w - e, 8)
                b_blk = xp_ref[pl.ds(jea, LD), :]
                b_rows = (pltpu.roll(b_blk, LD - e, 0) if e else b_blk)[0:BR]
                wrow = tab_ref[w, 0:WDIM]
                w_rows = jnp.broadcast_to(wrow[None, :], (BR, WDIM))
                return a_rows, b_rows, w_rows

            return f

        def boundary(w1):
            w2 = w1 + 1
            off2 = OFFS[w2]
            s1 = single(w1)

            def f():
                a1, b1, _ = s1()
                s_b = off2 - a
                c_blk = xp_ref[pl.ds(0, LD), :]
                a2 = pltpu.roll(c_blk, s_b, 0)[0:BR]
                b2 = pltpu.roll(c_blk, s_b + (LD - w2), 0)[0:BR]
                rows = lax.broadcasted_iota(jnp.int32, (BR, 1), 0)
                msk = rows >= s_b
                a_rows = jnp.where(msk, a2, a1)
                b_rows = jnp.where(msk, b2, b1)
                w1r = tab_ref[w1, 0:WDIM]
                w2r = tab_ref[w2, 0:WDIM]
                w_rows = jnp.where(
                    msk,
                    jnp.broadcast_to(w2r[None, :], (BR, WDIM)),
                    jnp.broadcast_to(w1r[None, :], (BR, WDIM)),
                )
                return a_rows, b_rows, w_rows

            return f

        w1 = _win_of(a)
        crosses = jnp.int32(0)
        for kk in range(1, NWIN):
            crosses = crosses | ((a < OFFS[kk]) & (a + BR > OFFS[kk])).astype(
                jnp.int32)

        branches = []
        for w in range(NWIN):
            branches.append(single(w))
            branches.append(boundary(w) if w < NWIN - 1 else single(w))
        a_rows, b_rows, w_rows = lax.switch(w1 * 2 + crosses, branches)

        out_ref[:, pl.ds(0, D)] = a_rows
        out_ref[:, pl.ds(D, D)] = b_rows
        out_ref[:, pl.ds(2 * D, WDIM)] = w_rows

    return pl.pallas_call(
        body,
        grid=(NBLK,),
        in_specs=[
            pl.BlockSpec((XPAD, D), lambda i: (0, 0)),
            pl.BlockSpec((NWIN, 128), lambda i: (0, 0)),
        ],
        out_specs=pl.BlockSpec((BR, OUTD), lambda i: (i, 0)),
        out_shape=jax.ShapeDtypeStruct((NSPAN, OUTD), jnp.float32),
    )


def kernel(x, span_width_embedding, batch_max_seq_len):
    del batch_max_seq_len
    x2 = x.reshape(SEQ, D)
    swe_pad = (
        jnp.zeros((span_width_embedding.shape[0], 128), span_width_embedding.dtype)
        .at[:, :WDIM]
        .set(span_width_embedding)
    )
    tab = _sc_lookup()(swe_pad)
    xp = jnp.pad(x2, ((0, XPAD - SEQ), (0, 0)))
    out = _tc_assemble()(xp, tab)
    return out.reshape(1, NSPAN, OUTD)


# speedup vs baseline: 1.0433x; 1.0128x over previous
import functools

import jax
import jax.numpy as jnp
from jax import lax
from jax.experimental import pallas as pl
from jax.experimental.pallas import tpu as pltpu
from jax.experimental.pallas import tpu_sc as plsc

SEQ = 2048
D = 768
WDIM = 20
NWIN = 8
NSPAN = NWIN * SEQ - (NWIN * (NWIN - 1)) // 2
OUTD = 2 * D + WDIM
R = 64
CHUNKS_PER_WORKER = 8


def _build():
    info = plsc.get_sparse_core_info()
    nc = info.num_cores

    mesh = plsc.VectorSubcoreMesh(core_axis_name="c", subcore_axis_name="s")

    @functools.partial(
        pl.kernel,
        mesh=mesh,
        out_type=jax.ShapeDtypeStruct((NSPAN, OUTD), jnp.float32),
        scratch_types=[
            pltpu.VMEM_SHARED((SEQ, D), jnp.float32),
            pltpu.VMEM((R, WDIM), jnp.float32),
            pltpu.SemaphoreType.DMA,
            pltpu.SemaphoreType.DMA,
        ],
        compiler_params=pltpu.CompilerParams(use_tc_tiling_on_sc=False),
    )
    def k(x_hbm, swe_hbm, out_hbm, xs, wbuf, sem, fill_sem):
        cid = lax.axis_index("c")
        sid = lax.axis_index("s")
        wid = sid * nc + cid
        wi = wid // 4
        q = wid % 4
        n = SEQ - wi
        off = SEQ * wi - (wi * (wi - 1)) // 2
        b = wi + 1 - (wi >= 5).astype(jnp.int32)

        fills = [
            pltpu.make_async_copy(
                swe_hbm.at[pl.ds(b, 1), :], wbuf.at[pl.ds(r, 1), :], fill_sem
            )
            for r in range(R)
        ]
        for cp in fills:
            cp.start()

        @pl.when(sid == 0)
        def _stage():
            pltpu.sync_copy(x_hbm, xs)

        plsc.subcore_barrier()

        copies = []
        for t in range(CHUNKS_PER_WORKER):
            c = q * CHUNKS_PER_WORKER + t
            j0 = jnp.minimum(c * R, n - R)
            j1 = j0 + wi
            r0 = off + j0
            copies.append(pltpu.make_async_copy(
                xs.at[pl.ds(j0, R), :],
                out_hbm.at[pl.ds(r0, R), pl.ds(0, D)], sem))
            copies.append(pltpu.make_async_copy(
                xs.at[pl.ds(j1, R), :],
                out_hbm.at[pl.ds(r0, R), pl.ds(D, D)], sem))
        for cp in copies:
            cp.start()
        for cp in fills:
            cp.wait()
        for t in range(CHUNKS_PER_WORKER):
            c = q * CHUNKS_PER_WORKER + t
            j0 = jnp.minimum(c * R, n - R)
            r0 = off + j0
            copies.append(pltpu.make_async_copy(
                wbuf, out_hbm.at[pl.ds(r0, R), pl.ds(2 * D, WDIM)], sem))
            copies[-1].start()
        for cp in copies:
            cp.wait()

    return k


def kernel(x, span_width_embedding, batch_max_seq_len):
    del batch_max_seq_len
    x2 = x.reshape(SEQ, D)
    out = _build()(x2, span_width_embedding)
    return out.reshape(1, NSPAN, OUTD)
